# baseline (speedup 1.0000x reference)
import functools

import jax
import jax.numpy as jnp
from jax import lax
from jax.experimental import pallas as pl
from jax.experimental.pallas import tpu as pltpu
from jax.experimental.pallas import tpu_sc as plsc

_TAU = 0.2
_K = 15
_CHUNK = 2048
_GRP = 128
_GPC = _CHUNK // _GRP
_NEG = -1.0e30


def _tc_body(ny, fx_ref, y_ref, sim_ref, g_ref):
    c = pl.program_id(1)
    fx = fx_ref[0]
    xn = jnp.sqrt(jnp.sum(fx * fx, axis=1, keepdims=True))
    fxn = fx / jnp.maximum(xn, 1e-12)
    y = y_ref[0]
    yn = jnp.sqrt(jnp.sum(y * y, axis=1, keepdims=True))
    fyn = y / jnp.maximum(yn, 1e-12)
    sim = lax.dot_general(
        fxn, fyn, (((1,), (1,)), ((), ())),
        preferred_element_type=jnp.float32,
    ) / _TAU
    col = c * _CHUNK + lax.broadcasted_iota(jnp.int32, sim.shape, 1)
    sim = jnp.where(col < ny, sim, _NEG)
    sim_ref[0] = sim
    parts = [
        jnp.max(sim[:, g * _GRP:(g + 1) * _GRP], axis=1, keepdims=True)
        for g in range(_GPC)
    ]
    g_ref[0, 0] = jnp.concatenate(parts, axis=1)


def _make_sc_kernel(nrows, nchunks, ngroups):
    rpw = nrows // 32
    mesh = plsc.VectorSubcoreMesh(core_axis_name="c", subcore_axis_name="s")
    imax = jnp.int32(2**31 - 1)

    @functools.partial(
        pl.kernel,
        out_type=[
            jax.ShapeDtypeStruct((nrows * 16,), jnp.float32),
            jax.ShapeDtypeStruct((nrows * 16,), jnp.int32),
        ],
        mesh=mesh,
        scratch_types=[
            pltpu.VMEM((nchunks * rpw * _GPC,), jnp.float32),
            pltpu.VMEM((16, _GRP), jnp.float32),
            pltpu.VMEM((16, _GRP), jnp.float32),
            pltpu.VMEM((rpw * 16,), jnp.float32),
            pltpu.VMEM((rpw * 16,), jnp.int32),
            pltpu.SemaphoreType.DMA,
            pltpu.SemaphoreType.DMA,
            pltpu.SemaphoreType.DMA,
        ],
        compiler_params=pltpu.CompilerParams(needs_layout_passes=False),
    )
    def sc_kernel(sim_hbm, g_hbm, outv_hbm, outc_hbm,
                  g_v, slab_a, slab_b, ov_v, oc_v, sem_g, sem_a, sem_b):
        wid = lax.axis_index("s") * 2 + lax.axis_index("c")
        row0 = wid * rpw
        b = row0 // 1024
        r0 = row0 % 1024
        lane = lax.iota(jnp.int32, 16)

        gsz = rpw * _GPC
        nxg = 1024 * _GPC
        copies = [
            pltpu.async_copy(
                g_hbm.at[pl.ds((b * nchunks + j) * nxg + r0 * _GPC, gsz)],
                g_v.at[pl.ds(j * gsz, gsz)], sem_g)
            for j in range(nchunks)
        ]
        for cp in copies:
            cp.wait()

        def merge16(rk, rv, nk, nv):
            nk, nv = plsc.sort_key_val(nk, nv, descending=True)
            nk = lax.rev(nk, (0,))
            nv = lax.rev(nv, (0,))
            m = rk >= nk
            mk = jnp.where(m, rk, nk)
            mv = jnp.where(m, rv, nv)
            ok, ov = plsc.sort_key_val(mk, mv, descending=True)
            return ok, ov

        neg_v = jnp.full((16,), _NEG, jnp.float32)

        def slot15(k):
            return jnp.zeros((16,), jnp.float32) + jnp.max(
                jnp.where(lane == 15, k, _NEG))

        def phase_a(r):
            def ga(j, ac):
                rk, rv, tm = ac
                gvals = g_v[pl.ds(j * gsz + r * _GPC, 16)]

                def do(_):
                    nk, nv = merge16(rk, rv, gvals, j * _GPC + lane)
                    return nk, nv, slot15(nk)

                return lax.cond(jnp.any(gvals > tm), do,
                                lambda _: (rk, rv, tm), 0)

            rk, rv, _ = lax.fori_loop(
                0, nchunks, ga,
                (neg_v, jnp.zeros((16,), jnp.int32), neg_v),
            )
            return rv

        def start_gather(r, rv, buf, sem):
            ids = (row0 + r) * ngroups + rv
            pltpu.async_copy(sim_hbm.at[ids], buf, sem)

        def wait_gather(buf, sem):
            ids0 = jnp.zeros((16,), jnp.int32)
            pltpu.make_async_copy(sim_hbm.at[ids0], buf, sem).wait()

        def consume(r, rv, buf):
            def gs(s, sc):
                gbase = jnp.sum(jnp.where(lane == s, rv, 0)) * _GRP

                svec = jnp.full((16,), 0, jnp.int32) + s

                def gv(v, vc):
                    ck, cv, tm = vc
                    off = v * 16 + lane
                    vals = plsc.load_gather(buf, [svec, off])

                    def do(_):
                        nk, nv = merge16(ck, cv, vals, gbase + off)
                        return nk, nv, slot15(nk)

                    return lax.cond(jnp.any(vals > tm), do,
                                    lambda _: (ck, cv, tm), 0)

                return lax.fori_loop(0, 8, gv, sc)

            ck, cv, _ = lax.fori_loop(
                0, 16, gs,
                (neg_v, jnp.zeros((16,), jnp.int32), neg_v),
            )

            vmax = jnp.max(ck)
            e = jnp.where(lane < _K, jnp.exp(ck - vmax), 0.0)
            sm = e / jnp.sum(e)

            keys = jnp.where(lane < _K, cv, imax)
            sk, sv = plsc.sort_key_val(keys, sm, descending=False)
            ov_v[pl.ds(r * 16, 16)] = sv
            oc_v[pl.ds(r * 16, 16)] = sk

        rv0 = phase_a(0)
        start_gather(0, rv0, slab_a, sem_a)

        def pair_body(p, rv_e):
            ro = 2 * p + 1
            rv_o = phase_a(ro)
            start_gather(ro, rv_o, slab_b, sem_b)
            wait_gather(slab_a, sem_a)
            consume(2 * p, rv_e, slab_a)
            re = jnp.minimum(2 * p + 2, rpw - 1)
            rv_e2 = phase_a(re)
            start_gather(re, rv_e2, slab_a, sem_a)
            wait_gather(slab_b, sem_b)
            consume(ro, rv_o, slab_b)
            return rv_e2

        lax.fori_loop(0, rpw // 2, pair_body, rv0)
        wait_gather(slab_a, sem_a)

        pltpu.sync_copy(ov_v, outv_hbm.at[pl.ds(row0 * 16, rpw * 16)])
        pltpu.sync_copy(oc_v, outc_hbm.at[pl.ds(row0 * 16, rpw * 16)])

    return sc_kernel


def kernel(feat_x, feat_y):
    B, Nx, C = feat_x.shape
    Ny = feat_y.shape[1]
    nchunks = -(-Ny // _CHUNK)
    nyp = nchunks * _CHUNK
    ngroups = nyp // _GRP
    nrows = B * Nx

    fyp = jnp.pad(feat_y, ((0, 0), (0, nyp - Ny), (0, 0)))

    sim, g = pl.pallas_call(
        functools.partial(_tc_body, Ny),
        grid=(B, nchunks),
        in_specs=[
            pl.BlockSpec((1, Nx, C), lambda b, c: (b, 0, 0)),
            pl.BlockSpec((1, _CHUNK, C), lambda b, c: (b, c, 0)),
        ],
        out_specs=[
            pl.BlockSpec((1, Nx, _CHUNK), lambda b, c: (b, 0, c)),
            pl.BlockSpec((1, 1, Nx, _GPC), lambda b, c: (b, c, 0, 0)),
        ],
        out_shape=[
            jax.ShapeDtypeStruct((B, Nx, nyp), jnp.float32),
            jax.ShapeDtypeStruct((B, nchunks, Nx, _GPC), jnp.float32),
        ],
    )(feat_x, fyp)

    sim_slabs = sim.reshape(nrows * ngroups, _GRP)
    g_flat = g.reshape(-1)
    outv, outc = _make_sc_kernel(nrows, nchunks, ngroups)(sim_slabs, g_flat)

    values = outv.reshape(nrows, 16)[:, :_K].reshape(-1)
    cols = outc.reshape(nrows, 16)[:, :_K].reshape(-1)
    bcol = jnp.repeat(jnp.arange(B, dtype=jnp.int32), Nx * _K)
    rows = jnp.tile(jnp.repeat(jnp.arange(Nx, dtype=jnp.int32), _K), B)
    indices = jnp.stack([bcol, rows, cols], axis=0)
    return indices, values


# speedup vs baseline: 1.0406x; 1.0406x over previous
import functools

import jax
import jax.numpy as jnp
from jax import lax
from jax.experimental import pallas as pl
from jax.experimental.pallas import tpu as pltpu
from jax.experimental.pallas import tpu_sc as plsc

_TAU = 0.2
_K = 15
_CHUNK = 2048
_GRP = 128
_GPC = _CHUNK // _GRP
_NEG = -1.0e30


def _tc_body(ny, fx_ref, y_ref, sim_ref, g_ref):
    c = pl.program_id(1)
    fx = fx_ref[0]
    xn = jnp.sqrt(jnp.sum(fx * fx, axis=1, keepdims=True))
    fxn = fx / jnp.maximum(xn, 1e-12)
    y = y_ref[0]
    yn = jnp.sqrt(jnp.sum(y * y, axis=1, keepdims=True))
    fyn = y / jnp.maximum(yn, 1e-12)
    sim = lax.dot_general(
        fxn, fyn, (((1,), (1,)), ((), ())),
        preferred_element_type=jnp.float32,
    ) / _TAU
    col = c * _CHUNK + lax.broadcasted_iota(jnp.int32, sim.shape, 1)
    sim = jnp.where(col < ny, sim, _NEG)
    sim_ref[0, 0] = sim
    parts = [
        jnp.max(sim[:, g * _GRP:(g + 1) * _GRP], axis=1, keepdims=True)
        for g in range(_GPC)
    ]
    g_ref[0, 0] = jnp.concatenate(parts, axis=1)


def _make_sc_kernel(nrows, nchunks, ngroups):
    rpw = nrows // 32
    mesh = plsc.VectorSubcoreMesh(core_axis_name="c", subcore_axis_name="s")
    imax = jnp.int32(2**31 - 1)

    @functools.partial(
        pl.kernel,
        out_type=[
            jax.ShapeDtypeStruct((nrows * 16,), jnp.float32),
            jax.ShapeDtypeStruct((nrows * 16,), jnp.int32),
        ],
        mesh=mesh,
        scratch_types=[
            pltpu.VMEM((nchunks * rpw * _GPC,), jnp.float32),
            pltpu.VMEM((16, _GRP), jnp.float32),
            pltpu.VMEM((16, _GRP), jnp.float32),
            pltpu.VMEM((rpw * 16,), jnp.float32),
            pltpu.VMEM((rpw * 16,), jnp.int32),
            pltpu.SemaphoreType.DMA,
            pltpu.SemaphoreType.DMA,
            pltpu.SemaphoreType.DMA,
        ],
        compiler_params=pltpu.CompilerParams(needs_layout_passes=False),
    )
    def sc_kernel(sim_hbm, g_hbm, outv_hbm, outc_hbm,
                  g_v, slab_a, slab_b, ov_v, oc_v, sem_g, sem_a, sem_b):
        wid = lax.axis_index("s") * 2 + lax.axis_index("c")
        row0 = wid * rpw
        b = row0 // 1024
        r0 = row0 % 1024
        lane = lax.iota(jnp.int32, 16)

        gsz = rpw * _GPC
        nxg = 1024 * _GPC
        copies = [
            pltpu.async_copy(
                g_hbm.at[pl.ds((b * nchunks + j) * nxg + r0 * _GPC, gsz)],
                g_v.at[pl.ds(j * gsz, gsz)], sem_g)
            for j in range(nchunks)
        ]
        for cp in copies:
            cp.wait()

        def merge16(rk, rv, nk, nv):
            nk, nv = plsc.sort_key_val(nk, nv, descending=True)
            nk = lax.rev(nk, (0,))
            nv = lax.rev(nv, (0,))
            m = rk >= nk
            mk = jnp.where(m, rk, nk)
            mv = jnp.where(m, rv, nv)
            ok, ov = plsc.sort_key_val(mk, mv, descending=True)
            return ok, ov

        neg_v = jnp.full((16,), _NEG, jnp.float32)

        def phase_a(r):
            def ga(j, ac):
                rk, rv = ac
                gvals = g_v[pl.ds(j * gsz + r * _GPC, 16)]
                return merge16(rk, rv, gvals, j * _GPC + lane)

            rk, rv = lax.fori_loop(
                0, nchunks, ga,
                (neg_v, jnp.zeros((16,), jnp.int32)),
            )
            return rv

        def slab_ids(r, rv):
            cc = lax.shift_right_logical(rv, 4)
            kk = lax.bitwise_and(rv, 15)
            return (b * nchunks + cc) * (1024 * _GPC) + (r0 + r) * _GPC + kk

        def start_gather(r, rv, buf, sem):
            pltpu.async_copy(sim_hbm.at[slab_ids(r, rv)], buf, sem)

        def wait_gather(buf, sem):
            ids0 = jnp.zeros((16,), jnp.int32)
            pltpu.make_async_copy(sim_hbm.at[ids0], buf, sem).wait()

        def consume(r, rv, buf):
            def gs(s, sc):
                gbase = jnp.sum(jnp.where(lane == s, rv, 0)) * _GRP

                svec = jnp.full((16,), 0, jnp.int32) + s

                def gv(v, vc):
                    ck, cv = vc
                    off = v * 16 + lane
                    vals = plsc.load_gather(buf, [svec, off])
                    return merge16(ck, cv, vals, gbase + off)

                return lax.fori_loop(0, 8, gv, sc)

            ck, cv = lax.fori_loop(
                0, 16, gs,
                (neg_v, jnp.zeros((16,), jnp.int32)),
            )

            vmax = jnp.max(ck)
            e = jnp.where(lane < _K, jnp.exp(ck - vmax), 0.0)
            sm = e / jnp.sum(e)

            keys = jnp.where(lane < _K, cv, imax)
            sk, sv = plsc.sort_key_val(keys, sm, descending=False)
            ov_v[pl.ds(r * 16, 16)] = sv
            oc_v[pl.ds(r * 16, 16)] = sk

        rv0 = phase_a(0)
        start_gather(0, rv0, slab_a, sem_a)

        def pair_body(p, rv_e):
            ro = 2 * p + 1
            rv_o = phase_a(ro)
            start_gather(ro, rv_o, slab_b, sem_b)
            wait_gather(slab_a, sem_a)
            consume(2 * p, rv_e, slab_a)
            re = jnp.minimum(2 * p + 2, rpw - 1)
            rv_e2 = phase_a(re)
            start_gather(re, rv_e2, slab_a, sem_a)
            wait_gather(slab_b, sem_b)
            consume(ro, rv_o, slab_b)
            return rv_e2

        lax.fori_loop(0, rpw // 2, pair_body, rv0)
        wait_gather(slab_a, sem_a)

        pltpu.sync_copy(ov_v, outv_hbm.at[pl.ds(row0 * 16, rpw * 16)])
        pltpu.sync_copy(oc_v, outc_hbm.at[pl.ds(row0 * 16, rpw * 16)])

    return sc_kernel


def kernel(feat_x, feat_y):
    B, Nx, C = feat_x.shape
    Ny = feat_y.shape[1]
    nchunks = -(-Ny // _CHUNK)
    nyp = nchunks * _CHUNK
    ngroups = nyp // _GRP
    nrows = B * Nx

    fyp = jnp.pad(feat_y, ((0, 0), (0, nyp - Ny), (0, 0)))

    sim, g = pl.pallas_call(
        functools.partial(_tc_body, Ny),
        grid=(B, nchunks),
        in_specs=[
            pl.BlockSpec((1, Nx, C), lambda b, c: (b, 0, 0)),
            pl.BlockSpec((1, _CHUNK, C), lambda b, c: (b, c, 0)),
        ],
        out_specs=[
            pl.BlockSpec((1, 1, Nx, _CHUNK), lambda b, c: (b, c, 0, 0)),
            pl.BlockSpec((1, 1, Nx, _GPC), lambda b, c: (b, c, 0, 0)),
        ],
        out_shape=[
            jax.ShapeDtypeStruct((B, nchunks, Nx, _CHUNK), jnp.float32),
            jax.ShapeDtypeStruct((B, nchunks, Nx, _GPC), jnp.float32),
        ],
    )(feat_x, fyp)

    sim_slabs = sim.reshape(-1, _GRP)
    g_flat = g.reshape(-1)
    outv, outc = _make_sc_kernel(nrows, nchunks, ngroups)(sim_slabs, g_flat)

    values = outv.reshape(nrows, 16)[:, :_K].reshape(-1)
    cols = outc.reshape(nrows, 16)[:, :_K].reshape(-1)
    bcol = jnp.repeat(jnp.arange(B, dtype=jnp.int32), Nx * _K)
    rows = jnp.tile(jnp.repeat(jnp.arange(Nx, dtype=jnp.int32), _K), B)
    indices = jnp.stack([bcol, rows, cols], axis=0)
    return indices, values


# speedup vs baseline: 2.4130x; 2.3189x over previous
import functools

import jax
import jax.numpy as jnp
from jax import lax
from jax.experimental import pallas as pl
from jax.experimental.pallas import tpu as pltpu
from jax.experimental.pallas import tpu_sc as plsc

_TAU = 0.2
_K = 15
_CHUNK = 2048
_GRP = 128
_GPC = _CHUNK // _GRP
_NEG = -1.0e30


def _tc_body(ny, fx_ref, y_ref, sim_ref, g_ref):
    c = pl.program_id(1)
    fx = fx_ref[0]
    xn = jnp.sqrt(jnp.sum(fx * fx, axis=1, keepdims=True))
    fxn = fx / jnp.maximum(xn, 1e-12)
    y = y_ref[0]
    yn = jnp.sqrt(jnp.sum(y * y, axis=1, keepdims=True))
    fyn = y / jnp.maximum(yn, 1e-12)
    sim = lax.dot_general(
        fxn, fyn, (((1,), (1,)), ((), ())),
        preferred_element_type=jnp.float32,
    ) / _TAU
    col = c * _CHUNK + lax.broadcasted_iota(jnp.int32, sim.shape, 1)
    sim = jnp.where(col < ny, sim, _NEG)
    nx = sim.shape[0]
    parts = []
    for g in range(_GPC):
        blk = sim[:, g * _GRP:(g + 1) * _GRP]
        sim_ref[pl.ds(g * nx, nx), :] = blk
        parts.append(jnp.max(blk, axis=1, keepdims=True))
    g_ref[0, 0] = jnp.concatenate(parts, axis=1)


def _make_sc_kernel(nrows, nchunks, ngroups):
    rpw = nrows // 32
    mesh = plsc.VectorSubcoreMesh(core_axis_name="c", subcore_axis_name="s")
    imax = jnp.int32(2**31 - 1)

    @functools.partial(
        pl.kernel,
        out_type=[
            jax.ShapeDtypeStruct((nrows * 16,), jnp.float32),
            jax.ShapeDtypeStruct((nrows * 16,), jnp.int32),
        ],
        mesh=mesh,
        scratch_types=[
            pltpu.VMEM((nchunks * rpw * _GPC,), jnp.float32),
            pltpu.VMEM((16, _GRP), jnp.float32),
            pltpu.VMEM((16, _GRP), jnp.float32),
            pltpu.VMEM((rpw * 16,), jnp.float32),
            pltpu.VMEM((rpw * 16,), jnp.int32),
            pltpu.SemaphoreType.DMA,
            pltpu.SemaphoreType.DMA,
            pltpu.SemaphoreType.DMA,
        ],
        compiler_params=pltpu.CompilerParams(needs_layout_passes=False),
    )
    def sc_kernel(sim_hbm, g_hbm, outv_hbm, outc_hbm,
                  g_v, slab_a, slab_b, ov_v, oc_v, sem_g, sem_a, sem_b):
        wid = lax.axis_index("s") * 2 + lax.axis_index("c")
        row0 = wid * rpw
        b = row0 // 1024
        r0 = row0 % 1024
        lane = lax.iota(jnp.int32, 16)

        gsz = rpw * _GPC
        nxg = 1024 * _GPC
        copies = [
            pltpu.async_copy(
                g_hbm.at[pl.ds((b * nchunks + j) * nxg + r0 * _GPC, gsz)],
                g_v.at[pl.ds(j * gsz, gsz)], sem_g)
            for j in range(nchunks)
        ]
        for cp in copies:
            cp.wait()

        def merge16(rk, rv, nk, nv):
            nk, nv = plsc.sort_key_val(nk, nv, descending=True)
            nk = lax.rev(nk, (0,))
            nv = lax.rev(nv, (0,))
            m = rk >= nk
            mk = jnp.where(m, rk, nk)
            mv = jnp.where(m, rv, nv)
            ok, ov = plsc.sort_key_val(mk, mv, descending=True)
            return ok, ov

        neg_v = jnp.full((16,), _NEG, jnp.float32)

        def phase_a(r):
            def ga(j, ac):
                rk, rv = ac
                gvals = g_v[pl.ds(j * gsz + r * _GPC, 16)]
                return merge16(rk, rv, gvals, j * _GPC + lane)

            rk, rv = lax.fori_loop(
                0, nchunks, ga,
                (neg_v, jnp.zeros((16,), jnp.int32)),
            )
            return rv

        def slab_ids(r, rv):
            cc = lax.shift_right_logical(rv, 4)
            kk = lax.bitwise_and(rv, 15)
            return ((b * nchunks + cc) * _GPC + kk) * 1024 + (r0 + r)

        def start_gather(r, rv, buf, sem):
            pltpu.async_copy(sim_hbm.at[slab_ids(r, rv)], buf, sem)

        def wait_gather(buf, sem):
            ids0 = jnp.zeros((16,), jnp.int32)
            pltpu.make_async_copy(sim_hbm.at[ids0], buf, sem).wait()

        def consume(r, rv, buf):
            def gs(s, sc):
                gbase = jnp.sum(jnp.where(lane == s, rv, 0)) * _GRP

                svec = jnp.full((16,), 0, jnp.int32) + s

                def gv(v, vc):
                    ck, cv = vc
                    off = v * 16 + lane
                    vals = plsc.load_gather(buf, [svec, off])
                    return merge16(ck, cv, vals, gbase + off)

                return lax.fori_loop(0, 8, gv, sc)

            ck, cv = lax.fori_loop(
                0, 16, gs,
                (neg_v, jnp.zeros((16,), jnp.int32)),
            )

            vmax = jnp.max(ck)
            e = jnp.where(lane < _K, jnp.exp(ck - vmax), 0.0)
            sm = e / jnp.sum(e)

            keys = jnp.where(lane < _K, cv, imax)
            sk, sv = plsc.sort_key_val(keys, sm, descending=False)
            ov_v[pl.ds(r * 16, 16)] = sv
            oc_v[pl.ds(r * 16, 16)] = sk

        rv0 = phase_a(0)
        start_gather(0, rv0, slab_a, sem_a)

        def pair_body(p, rv_e):
            ro = 2 * p + 1
            rv_o = phase_a(ro)
            start_gather(ro, rv_o, slab_b, sem_b)
            wait_gather(slab_a, sem_a)
            consume(2 * p, rv_e, slab_a)
            re = jnp.minimum(2 * p + 2, rpw - 1)
            rv_e2 = phase_a(re)
            start_gather(re, rv_e2, slab_a, sem_a)
            wait_gather(slab_b, sem_b)
            consume(ro, rv_o, slab_b)
            return rv_e2

        lax.fori_loop(0, rpw // 2, pair_body, rv0)
        wait_gather(slab_a, sem_a)

        pltpu.sync_copy(ov_v, outv_hbm.at[pl.ds(row0 * 16, rpw * 16)])
        pltpu.sync_copy(oc_v, outc_hbm.at[pl.ds(row0 * 16, rpw * 16)])

    return sc_kernel


def kernel(feat_x, feat_y):
    B, Nx, C = feat_x.shape
    Ny = feat_y.shape[1]
    nchunks = -(-Ny // _CHUNK)
    nyp = nchunks * _CHUNK
    ngroups = nyp // _GRP
    nrows = B * Nx

    fyp = jnp.pad(feat_y, ((0, 0), (0, nyp - Ny), (0, 0)))

    sim, g = pl.pallas_call(
        functools.partial(_tc_body, Ny),
        grid=(B, nchunks),
        in_specs=[
            pl.BlockSpec((1, Nx, C), lambda b, c: (b, 0, 0)),
            pl.BlockSpec((1, _CHUNK, C), lambda b, c: (b, c, 0)),
        ],
        out_specs=[
            pl.BlockSpec((Nx * _GPC, _GRP), lambda b, c: (b * nchunks + c, 0)),
            pl.BlockSpec((1, 1, Nx, _GPC), lambda b, c: (b, c, 0, 0)),
        ],
        out_shape=[
            jax.ShapeDtypeStruct((B * nchunks * Nx * _GPC, _GRP), jnp.float32),
            jax.ShapeDtypeStruct((B, nchunks, Nx, _GPC), jnp.float32),
        ],
    )(feat_x, fyp)

    sim_slabs = sim
    g_flat = g.reshape(-1)
    outv, outc = _make_sc_kernel(nrows, nchunks, ngroups)(sim_slabs, g_flat)

    values = outv.reshape(nrows, 16)[:, :_K].reshape(-1)
    cols = outc.reshape(nrows, 16)[:, :_K].reshape(-1)
    bcol = jnp.repeat(jnp.arange(B, dtype=jnp.int32), Nx * _K)
    rows = jnp.tile(jnp.repeat(jnp.arange(Nx, dtype=jnp.int32), _K), B)
    indices = jnp.stack([bcol, rows, cols], axis=0)
    return indices, values


# speedup vs baseline: 2.8599x; 1.1852x over previous
import functools

import jax
import jax.numpy as jnp
from jax import lax
from jax.experimental import pallas as pl
from jax.experimental.pallas import tpu as pltpu
from jax.experimental.pallas import tpu_sc as plsc

_TAU = 0.2
_K = 15
_CHUNK = 2048
_GRP = 128
_GPC = _CHUNK // _GRP
_NEG = -1.0e30


def _tc_body(ny, fx_ref, y_ref, sim_ref, g_ref):
    c = pl.program_id(0)
    fx = fx_ref[...]
    xn = jnp.sqrt(jnp.sum(fx * fx, axis=1, keepdims=True))
    fxn = fx / jnp.maximum(xn, 1e-12)
    y = y_ref[...]
    yn = jnp.sqrt(jnp.sum(y * y, axis=1, keepdims=True))
    fyn = y / jnp.maximum(yn, 1e-12)
    sim = lax.dot_general(
        fxn, fyn, (((1,), (1,)), ((), ())),
        preferred_element_type=jnp.float32,
    ) / _TAU
    col = c * _CHUNK + lax.broadcasted_iota(jnp.int32, sim.shape, 1)
    sim = jnp.where(col < ny, sim, _NEG)
    nx = sim.shape[0]
    parts = []
    for g in range(_GPC):
        blk = sim[:, g * _GRP:(g + 1) * _GRP]
        sim_ref[pl.ds(g * nx, nx), :] = blk
        parts.append(jnp.max(blk, axis=1, keepdims=True))
    g_ref[0] = jnp.concatenate(parts, axis=1)


def _make_sc_kernel(nrows, nchunks, ngroups):
    rpw = nrows // 32
    mesh = plsc.VectorSubcoreMesh(core_axis_name="c", subcore_axis_name="s")
    imax = jnp.int32(2**31 - 1)

    @functools.partial(
        pl.kernel,
        out_type=[
            jax.ShapeDtypeStruct((nrows * 16,), jnp.float32),
            jax.ShapeDtypeStruct((nrows * 16,), jnp.int32),
        ],
        mesh=mesh,
        scratch_types=[
            pltpu.VMEM((nchunks * rpw * _GPC,), jnp.float32),
            pltpu.VMEM((16, _GRP), jnp.float32),
            pltpu.VMEM((16, _GRP), jnp.float32),
            pltpu.VMEM((rpw * 16,), jnp.float32),
            pltpu.VMEM((rpw * 16,), jnp.int32),
            pltpu.SemaphoreType.DMA,
            pltpu.SemaphoreType.DMA,
            pltpu.SemaphoreType.DMA,
        ],
        compiler_params=pltpu.CompilerParams(needs_layout_passes=False),
    )
    def sc_kernel(sim_hbm, g_hbm, outv_hbm, outc_hbm,
                  g_v, slab_a, slab_b, ov_v, oc_v, sem_g, sem_a, sem_b):
        wid = lax.axis_index("s") * 2 + lax.axis_index("c")
        r0 = wid * rpw
        lane = lax.iota(jnp.int32, 16)

        gsz = rpw * _GPC
        nxg = nrows * _GPC
        copies = [
            pltpu.async_copy(
                g_hbm.at[pl.ds(j * nxg + r0 * _GPC, gsz)],
                g_v.at[pl.ds(j * gsz, gsz)], sem_g)
            for j in range(nchunks)
        ]
        for cp in copies:
            cp.wait()

        def merge16(rk, rv, nk, nv):
            nk, nv = plsc.sort_key_val(nk, nv, descending=True)
            nk = lax.rev(nk, (0,))
            nv = lax.rev(nv, (0,))
            m = rk >= nk
            mk = jnp.where(m, rk, nk)
            mv = jnp.where(m, rv, nv)
            ok, ov = plsc.sort_key_val(mk, mv, descending=True)
            return ok, ov

        neg_v = jnp.full((16,), _NEG, jnp.float32)

        def phase_a(r):
            def ga(j, ac):
                rk, rv = ac
                gvals = g_v[pl.ds(j * gsz + r * _GPC, 16)]
                return merge16(rk, rv, gvals, j * _GPC + lane)

            rk, rv = lax.fori_loop(
                0, nchunks, ga,
                (neg_v, jnp.zeros((16,), jnp.int32)),
            )
            return rv

        def slab_ids(r, rv):
            cc = lax.shift_right_logical(rv, 4)
            kk = lax.bitwise_and(rv, 15)
            return (cc * _GPC + kk) * nrows + (r0 + r)

        def start_gather(r, rv, buf, sem):
            pltpu.async_copy(sim_hbm.at[slab_ids(r, rv)], buf, sem)

        def wait_gather(buf, sem):
            ids0 = jnp.zeros((16,), jnp.int32)
            pltpu.make_async_copy(sim_hbm.at[ids0], buf, sem).wait()

        def consume(r, rv, buf):
            def gs(s, sc):
                gbase = jnp.sum(jnp.where(lane == s, rv, 0)) * _GRP

                svec = jnp.full((16,), 0, jnp.int32) + s

                def gv(v, vc):
                    ck, cv = vc
                    off = v * 16 + lane
                    vals = plsc.load_gather(buf, [svec, off])
                    return merge16(ck, cv, vals, gbase + off)

                return lax.fori_loop(0, 8, gv, sc)

            ck, cv = lax.fori_loop(
                0, 16, gs,
                (neg_v, jnp.zeros((16,), jnp.int32)),
            )

            vmax = jnp.max(ck)
            e = jnp.where(lane < _K, jnp.exp(ck - vmax), 0.0)
            sm = e / jnp.sum(e)

            keys = jnp.where(lane < _K, cv, imax)
            sk, sv = plsc.sort_key_val(keys, sm, descending=False)
            ov_v[pl.ds(r * 16, 16)] = sv
            oc_v[pl.ds(r * 16, 16)] = sk

        rv0 = phase_a(0)
        start_gather(0, rv0, slab_a, sem_a)

        def pair_body(p, rv_e):
            ro = 2 * p + 1
            rv_o = phase_a(ro)
            start_gather(ro, rv_o, slab_b, sem_b)
            wait_gather(slab_a, sem_a)
            consume(2 * p, rv_e, slab_a)
            re = jnp.minimum(2 * p + 2, rpw - 1)
            rv_e2 = phase_a(re)
            start_gather(re, rv_e2, slab_a, sem_a)
            wait_gather(slab_b, sem_b)
            consume(ro, rv_o, slab_b)
            return rv_e2

        lax.fori_loop(0, rpw // 2, pair_body, rv0)
        wait_gather(slab_a, sem_a)

        pltpu.sync_copy(ov_v, outv_hbm.at[pl.ds(r0 * 16, rpw * 16)])
        pltpu.sync_copy(oc_v, outc_hbm.at[pl.ds(r0 * 16, rpw * 16)])

    return sc_kernel


def kernel(feat_x, feat_y):
    B, Nx, C = feat_x.shape
    Ny = feat_y.shape[1]
    nchunks = -(-Ny // _CHUNK)
    nyp = nchunks * _CHUNK
    ngroups = nyp // _GRP
    nrows = B * Nx

    fyp = jnp.pad(feat_y, ((0, 0), (0, nyp - Ny), (0, 0)))

    tc = pl.pallas_call(
        functools.partial(_tc_body, Ny),
        grid=(nchunks,),
        in_specs=[
            pl.BlockSpec((Nx, C), lambda c: (0, 0)),
            pl.BlockSpec((_CHUNK, C), lambda c: (c, 0)),
        ],
        out_specs=[
            pl.BlockSpec((Nx * _GPC, _GRP), lambda c: (c, 0)),
            pl.BlockSpec((1, Nx, _GPC), lambda c: (c, 0, 0)),
        ],
        out_shape=[
            jax.ShapeDtypeStruct((nchunks * Nx * _GPC, _GRP), jnp.float32),
            jax.ShapeDtypeStruct((nchunks, Nx, _GPC), jnp.float32),
        ],
    )
    sc = _make_sc_kernel(Nx, nchunks, ngroups)

    outs = []
    for bi in range(B):
        sim_b, g_b = tc(feat_x[bi], fyp[bi])
        outs.append(sc(sim_b, g_b.reshape(-1)))
    outv = jnp.concatenate([o[0] for o in outs])
    outc = jnp.concatenate([o[1] for o in outs])

    values = outv.reshape(nrows, 16)[:, :_K].reshape(-1)
    cols = outc.reshape(nrows, 16)[:, :_K].reshape(-1)
    bcol = jnp.repeat(jnp.arange(B, dtype=jnp.int32), Nx * _K)
    rows = jnp.tile(jnp.repeat(jnp.arange(Nx, dtype=jnp.int32), _K), B)
    indices = jnp.stack([bcol, rows, cols], axis=0)
    return indices, values


# speedup vs baseline: 2.9296x; 1.0244x over previous
import functools

import jax
import jax.numpy as jnp
from jax import lax
from jax.experimental import pallas as pl
from jax.experimental.pallas import tpu as pltpu
from jax.experimental.pallas import tpu_sc as plsc

_TAU = 0.2
_K = 15
_CHUNK = 4096
_GRP = 128
_GPC = _CHUNK // _GRP
_CSH = _GPC.bit_length() - 1
_GSH = (_GPC // 16).bit_length() - 1
_GSM = _GPC // 16 - 1
_NEG = -1.0e30


def _tc_body(ny, fx_ref, y_ref, sim_ref, g_ref):
    c = pl.program_id(0)
    fx = fx_ref[...]
    xn = jnp.sqrt(jnp.sum(fx * fx, axis=1, keepdims=True))
    fxn = fx / jnp.maximum(xn, 1e-12)
    y = y_ref[...]
    yn = jnp.sqrt(jnp.sum(y * y, axis=1, keepdims=True))
    fyn = y / jnp.maximum(yn, 1e-12)
    sim = lax.dot_general(
        fxn, fyn, (((1,), (1,)), ((), ())),
        preferred_element_type=jnp.float32,
    ) / _TAU
    col = c * _CHUNK + lax.broadcasted_iota(jnp.int32, sim.shape, 1)
    sim = jnp.where(col < ny, sim, _NEG)
    nx = sim.shape[0]
    parts = []
    for g in range(_GPC):
        blk = sim[:, g * _GRP:(g + 1) * _GRP]
        sim_ref[pl.ds(g * nx, nx), :] = blk
        parts.append(jnp.max(blk, axis=1, keepdims=True))
    g_ref[0] = jnp.concatenate(parts, axis=1)


def _make_sc_kernel(nrows, nchunks, ngroups):
    rpw = nrows // 32
    mesh = plsc.VectorSubcoreMesh(core_axis_name="c", subcore_axis_name="s")
    imax = jnp.int32(2**31 - 1)

    @functools.partial(
        pl.kernel,
        out_type=[
            jax.ShapeDtypeStruct((nrows * 16,), jnp.float32),
            jax.ShapeDtypeStruct((nrows * 16,), jnp.int32),
        ],
        mesh=mesh,
        scratch_types=[
            pltpu.VMEM((nchunks * rpw * _GPC,), jnp.float32),
            pltpu.VMEM((16, _GRP), jnp.float32),
            pltpu.VMEM((16, _GRP), jnp.float32),
            pltpu.VMEM((rpw * 16,), jnp.float32),
            pltpu.VMEM((rpw * 16,), jnp.int32),
            pltpu.SemaphoreType.DMA,
            pltpu.SemaphoreType.DMA,
            pltpu.SemaphoreType.DMA,
        ],
        compiler_params=pltpu.CompilerParams(needs_layout_passes=False),
    )
    def sc_kernel(sim_hbm, g_hbm, outv_hbm, outc_hbm,
                  g_v, slab_a, slab_b, ov_v, oc_v, sem_g, sem_a, sem_b):
        wid = lax.axis_index("s") * 2 + lax.axis_index("c")
        r0 = wid * rpw
        lane = lax.iota(jnp.int32, 16)

        gsz = rpw * _GPC
        nxg = nrows * _GPC
        copies = [
            pltpu.async_copy(
                g_hbm.at[pl.ds(j * nxg + r0 * _GPC, gsz)],
                g_v.at[pl.ds(j * gsz, gsz)], sem_g)
            for j in range(nchunks)
        ]
        for cp in copies:
            cp.wait()

        def merge16(rk, rv, nk, nv):
            nk, nv = plsc.sort_key_val(nk, nv, descending=True)
            nk = lax.rev(nk, (0,))
            nv = lax.rev(nv, (0,))
            m = rk >= nk
            mk = jnp.where(m, rk, nk)
            mv = jnp.where(m, rv, nv)
            ok, ov = plsc.sort_key_val(mk, mv, descending=True)
            return ok, ov

        neg_v = jnp.full((16,), _NEG, jnp.float32)

        def phase_a(r):
            def ga(q, ac):
                rk, rv = ac
                j = lax.shift_right_logical(q, _GSH)
                h = lax.bitwise_and(q, _GSM)
                gvals = g_v[pl.ds(j * gsz + r * _GPC + h * 16, 16)]
                return merge16(rk, rv, gvals, q * 16 + lane)

            rk, rv = lax.fori_loop(
                0, nchunks * (_GPC // 16), ga,
                (neg_v, jnp.zeros((16,), jnp.int32)),
            )
            return rv

        def slab_ids(r, rv):
            cc = lax.shift_right_logical(rv, _CSH)
            kk = lax.bitwise_and(rv, _GPC - 1)
            return (cc * _GPC + kk) * nrows + (r0 + r)

        def start_gather(r, rv, buf, sem):
            pltpu.async_copy(sim_hbm.at[slab_ids(r, rv)], buf, sem)

        def wait_gather(buf, sem):
            ids0 = jnp.zeros((16,), jnp.int32)
            pltpu.make_async_copy(sim_hbm.at[ids0], buf, sem).wait()

        def consume(r, rv, buf):
            def gs(s, sc):
                gbase = jnp.sum(jnp.where(lane == s, rv, 0)) * _GRP

                svec = jnp.full((16,), 0, jnp.int32) + s

                def gv(v, vc):
                    ck, cv = vc
                    off = v * 16 + lane
                    vals = plsc.load_gather(buf, [svec, off])
                    return merge16(ck, cv, vals, gbase + off)

                return lax.fori_loop(0, 8, gv, sc)

            ck, cv = lax.fori_loop(
                0, 16, gs,
                (neg_v, jnp.zeros((16,), jnp.int32)),
            )

            vmax = jnp.max(ck)
            e = jnp.where(lane < _K, jnp.exp(ck - vmax), 0.0)
            sm = e / jnp.sum(e)

            keys = jnp.where(lane < _K, cv, imax)
            sk, sv = plsc.sort_key_val(keys, sm, descending=False)
            ov_v[pl.ds(r * 16, 16)] = sv
            oc_v[pl.ds(r * 16, 16)] = sk

        rv0 = phase_a(0)
        start_gather(0, rv0, slab_a, sem_a)

        def pair_body(p, rv_e):
            ro = 2 * p + 1
            rv_o = phase_a(ro)
            start_gather(ro, rv_o, slab_b, sem_b)
            wait_gather(slab_a, sem_a)
            consume(2 * p, rv_e, slab_a)
            re = jnp.minimum(2 * p + 2, rpw - 1)
            rv_e2 = phase_a(re)
            start_gather(re, rv_e2, slab_a, sem_a)
            wait_gather(slab_b, sem_b)
            consume(ro, rv_o, slab_b)
            return rv_e2

        lax.fori_loop(0, rpw // 2, pair_body, rv0)
        wait_gather(slab_a, sem_a)

        pltpu.sync_copy(ov_v, outv_hbm.at[pl.ds(r0 * 16, rpw * 16)])
        pltpu.sync_copy(oc_v, outc_hbm.at[pl.ds(r0 * 16, rpw * 16)])

    return sc_kernel


def kernel(feat_x, feat_y):
    B, Nx, C = feat_x.shape
    Ny = feat_y.shape[1]
    nchunks = -(-Ny // _CHUNK)
    nyp = nchunks * _CHUNK
    ngroups = nyp // _GRP
    nrows = B * Nx

    fyp = jnp.pad(feat_y, ((0, 0), (0, nyp - Ny), (0, 0)))

    tc = pl.pallas_call(
        functools.partial(_tc_body, Ny),
        grid=(nchunks,),
        in_specs=[
            pl.BlockSpec((Nx, C), lambda c: (0, 0)),
            pl.BlockSpec((_CHUNK, C), lambda c: (c, 0)),
        ],
        out_specs=[
            pl.BlockSpec((Nx * _GPC, _GRP), lambda c: (c, 0)),
            pl.BlockSpec((1, Nx, _GPC), lambda c: (c, 0, 0)),
        ],
        out_shape=[
            jax.ShapeDtypeStruct((nchunks * Nx * _GPC, _GRP), jnp.float32),
            jax.ShapeDtypeStruct((nchunks, Nx, _GPC), jnp.float32),
        ],
    )
    sc = _make_sc_kernel(Nx, nchunks, ngroups)

    outs = []
    for bi in range(B):
        sim_b, g_b = tc(feat_x[bi], fyp[bi])
        outs.append(sc(sim_b, g_b.reshape(-1)))
    outv = jnp.concatenate([o[0] for o in outs])
    outc = jnp.concatenate([o[1] for o in outs])

    values = outv.reshape(nrows, 16)[:, :_K].reshape(-1)
    cols = outc.reshape(nrows, 16)[:, :_K].reshape(-1)
    bcol = jnp.repeat(jnp.arange(B, dtype=jnp.int32), Nx * _K)
    rows = jnp.tile(jnp.repeat(jnp.arange(Nx, dtype=jnp.int32), _K), B)
    indices = jnp.stack([bcol, rows, cols], axis=0)
    return indices, values
